# baseline (speedup 1.0000x reference)
import jax
import jax.numpy as jnp
from jax.experimental import pallas as pl
from jax.experimental.pallas import tpu as pltpu

D = 2048
F = 8192
E = 8
K = 2
T = 2048

BM = 640
BN = 128
NCHUNK = F // BN
NB = (K * T) // BM + E
NROWS = NB * BM


def _router_body(x_ref, rw_ref, ids_ref, wts_ref, aux_ref):
    x = x_ref[...]
    rw = rw_ref[...]
    logits = jax.lax.dot_general(
        x, rw, (((1,), (1,)), ((), ())),
        preferred_element_type=jnp.float32)
    mx = jnp.max(logits, axis=1, keepdims=True)
    ex = jnp.exp(logits - mx)
    scores = ex / jnp.sum(ex, axis=1, keepdims=True)

    idx = jax.lax.broadcasted_iota(jnp.int32, scores.shape, 1)
    m1 = jnp.max(scores, axis=1, keepdims=True)
    i1 = jnp.min(jnp.where(scores == m1, idx, E), axis=1, keepdims=True)
    masked = jnp.where(idx == i1, -jnp.float32(1e30), scores)
    m2 = jnp.max(masked, axis=1, keepdims=True)
    i2 = jnp.min(jnp.where(masked == m2, idx, E), axis=1, keepdims=True)

    s = m1 + m2
    ids_ref[...] = jnp.concatenate([i1, i2], axis=1).astype(jnp.int32)
    wts_ref[...] = jnp.concatenate([m1 / s, m2 / s], axis=1)

    sel = (idx == i1) | (idx == i2)
    cnt = jnp.sum(sel.astype(jnp.float32), axis=0, keepdims=True)
    probs = jnp.sum(scores, axis=0, keepdims=True) / jnp.float32(T)
    aux_ref[...] = jnp.float32(E) * jnp.sum(
        probs * cnt / jnp.float32(T), axis=1, keepdims=True)


def _router(xf, router_w):
    return pl.pallas_call(
        _router_body,
        out_shape=[
            jax.ShapeDtypeStruct((T, K), jnp.int32),
            jax.ShapeDtypeStruct((T, K), jnp.float32),
            jax.ShapeDtypeStruct((1, 1), jnp.float32),
        ],
    )(xf, router_w)


def _gmm_body(be_ref, ba_ref, tid_ref, wt_ref, x_ref, g_ref, u_ref, d_ref,
              out_ref, xg_ref, acc_ref):
    m = pl.program_id(0)
    n = pl.program_id(1)
    active = ba_ref[m] > 0

    @pl.when((m == 0) & (n == 0))
    def _():
        out_ref[...] = jnp.zeros((T, D), jnp.float32)

    @pl.when(active & (n == 0))
    def _():
        def gather(i, c):
            t = tid_ref[m * BM + i]
            xg_ref[pl.ds(i, 1), :] = x_ref[pl.ds(t, 1), :]
            return c
        jax.lax.fori_loop(0, BM, gather, 0)

    @pl.when(active)
    def _():
        xg = xg_ref[...]
        g = jax.lax.dot_general(xg, g_ref[0], (((1,), (1,)), ((), ())),
                                preferred_element_type=jnp.float32)
        u = jax.lax.dot_general(xg, u_ref[0], (((1,), (1,)), ((), ())),
                                preferred_element_type=jnp.float32)
        h = (g * jax.lax.logistic(g)) * u
        y = jax.lax.dot_general(h, d_ref[0], (((1,), (1,)), ((), ())),
                                preferred_element_type=jnp.float32)

        @pl.when(n == 0)
        def _():
            acc_ref[...] = y

        @pl.when(n > 0)
        def _():
            acc_ref[...] = acc_ref[...] + y

    @pl.when(active & (n == NCHUNK - 1))
    def _():
        def scatter(i, c):
            t = tid_ref[m * BM + i]
            w = wt_ref[m * BM + i]
            out_ref[pl.ds(t, 1), :] = (out_ref[pl.ds(t, 1), :]
                                       + w * acc_ref[pl.ds(i, 1), :])
            return c
        jax.lax.fori_loop(0, BM, scatter, 0)


def _gmm(xf, gate_w, up_w, down_w, block_expert, block_active, token_ids,
         row_wts):
    grid_spec = pltpu.PrefetchScalarGridSpec(
        num_scalar_prefetch=4,
        grid=(NB, NCHUNK),
        in_specs=[
            pl.BlockSpec((T, D), lambda m, n, be, ba, tid, wt: (0, 0)),
            pl.BlockSpec((1, BN, D), lambda m, n, be, ba, tid, wt: (be[m], n, 0)),
            pl.BlockSpec((1, BN, D), lambda m, n, be, ba, tid, wt: (be[m], n, 0)),
            pl.BlockSpec((1, D, BN), lambda m, n, be, ba, tid, wt: (be[m], 0, n)),
        ],
        out_specs=pl.BlockSpec((T, D), lambda m, n, be, ba, tid, wt: (0, 0)),
        scratch_shapes=[
            pltpu.VMEM((BM, D), jnp.float32),
            pltpu.VMEM((BM, D), jnp.float32),
        ],
    )
    return pl.pallas_call(
        _gmm_body,
        grid_spec=grid_spec,
        out_shape=jax.ShapeDtypeStruct((T, D), jnp.float32),
        compiler_params=pltpu.CompilerParams(
            dimension_semantics=("arbitrary", "arbitrary")),
    )(block_expert, block_active, token_ids, row_wts, xf, gate_w, up_w,
      down_w)


def kernel(x, router_w, gate_w, up_w, down_w):
    Bq, Tq, Dq = x.shape
    xf = x.reshape(Tq, Dq)

    ids, wts, aux = _router(xf, router_w)

    e_flat = ids.reshape(-1)
    onehot = jax.nn.one_hot(e_flat, E, dtype=jnp.int32)
    ranks_excl = jnp.cumsum(onehot, axis=0) - onehot
    rank = jnp.sum(ranks_excl * onehot, axis=1)
    counts = jnp.sum(onehot, axis=0)
    padded = ((counts + BM - 1) // BM) * BM
    ends = jnp.cumsum(padded)
    p_off = ends - padded
    pos = jnp.sum(onehot * p_off[None, :], axis=1) + rank
    tok = jnp.arange(K * T, dtype=jnp.int32) // K
    token_ids = jnp.zeros((NROWS,), jnp.int32).at[pos].set(tok)
    row_wts = jnp.zeros((NROWS,), jnp.float32).at[pos].set(wts.reshape(-1))
    total = ends[E - 1]
    starts = jnp.arange(NB, dtype=jnp.int32) * BM
    block_active = (starts < total).astype(jnp.int32)
    block_expert = jnp.minimum(
        jnp.sum((starts[:, None] >= ends[None, :]).astype(jnp.int32), axis=1),
        E - 1).astype(jnp.int32)

    out = _gmm(xf, gate_w, up_w, down_w, block_expert, block_active,
               token_ids, row_wts)

    return out.reshape(Bq, Tq, Dq), aux[0, 0]


# speedup vs baseline: 1.3839x; 1.3839x over previous
import jax
import jax.numpy as jnp
from jax.experimental import pallas as pl
from jax.experimental.pallas import tpu as pltpu

D = 2048
F = 8192
E = 8
K = 2
T = 2048

BM = 512
BN = 256
NCHUNK = F // BN
NB = (K * T) // BM + E
NROWS = NB * BM


def _router_body(x_ref, rw_ref, ids_ref, wts_ref, aux_ref):
    x = x_ref[...]
    rw = rw_ref[...]
    logits = jax.lax.dot_general(
        x, rw, (((1,), (1,)), ((), ())),
        preferred_element_type=jnp.float32)
    mx = jnp.max(logits, axis=1, keepdims=True)
    ex = jnp.exp(logits - mx)
    scores = ex / jnp.sum(ex, axis=1, keepdims=True)

    idx = jax.lax.broadcasted_iota(jnp.int32, scores.shape, 1)
    m1 = jnp.max(scores, axis=1, keepdims=True)
    i1 = jnp.min(jnp.where(scores == m1, idx, E), axis=1, keepdims=True)
    masked = jnp.where(idx == i1, -jnp.float32(1e30), scores)
    m2 = jnp.max(masked, axis=1, keepdims=True)
    i2 = jnp.min(jnp.where(masked == m2, idx, E), axis=1, keepdims=True)

    s = m1 + m2
    ids_ref[...] = jnp.concatenate([i1, i2], axis=1).astype(jnp.int32)
    wts_ref[...] = jnp.concatenate([m1 / s, m2 / s], axis=1)

    sel = (idx == i1) | (idx == i2)
    cnt = jnp.sum(sel.astype(jnp.float32), axis=0, keepdims=True)
    probs = jnp.sum(scores, axis=0, keepdims=True) / jnp.float32(T)
    aux_ref[...] = jnp.float32(E) * jnp.sum(
        probs * cnt / jnp.float32(T), axis=1, keepdims=True)


def _router(xf, router_w):
    return pl.pallas_call(
        _router_body,
        out_shape=[
            jax.ShapeDtypeStruct((T, K), jnp.int32),
            jax.ShapeDtypeStruct((T, K), jnp.float32),
            jax.ShapeDtypeStruct((1, 1), jnp.float32),
        ],
    )(xf, router_w)


def _gmm_body(be_ref, ba_ref, tid_ref, wt_ref, x_ref, g_ref, u_ref, d_ref,
              out_ref, xg_ref, acc_ref):
    m = pl.program_id(0)
    n = pl.program_id(1)
    active = ba_ref[m] > 0

    @pl.when((m == 0) & (n == 0))
    def _():
        out_ref[...] = jnp.zeros((T, D), jnp.float32)

    @pl.when(active & (n == 0))
    def _():
        def gather(i, c):
            t = tid_ref[m * BM + i]
            xg_ref[pl.ds(i, 1), :] = x_ref[pl.ds(t, 1), :]
            return c
        jax.lax.fori_loop(0, BM, gather, 0)

    @pl.when(active)
    def _():
        xg = xg_ref[...]
        g = jax.lax.dot_general(xg, g_ref[0], (((1,), (1,)), ((), ())),
                                preferred_element_type=jnp.float32)
        u = jax.lax.dot_general(xg, u_ref[0], (((1,), (1,)), ((), ())),
                                preferred_element_type=jnp.float32)
        h = (g * jax.lax.logistic(g)) * u
        y = jax.lax.dot_general(h, d_ref[0], (((1,), (1,)), ((), ())),
                                preferred_element_type=jnp.float32)

        @pl.when(n == 0)
        def _():
            acc_ref[...] = y

        @pl.when(n > 0)
        def _():
            acc_ref[...] = acc_ref[...] + y

    @pl.when(active & (n == NCHUNK - 1))
    def _():
        def scatter(i, c):
            t = tid_ref[m * BM + i]
            w = wt_ref[m * BM + i]
            out_ref[pl.ds(t, 1), :] = (out_ref[pl.ds(t, 1), :]
                                       + w * acc_ref[pl.ds(i, 1), :])
            return c
        jax.lax.fori_loop(0, BM, scatter, 0)


def _gmm(xf, gate_w, up_w, down_w, block_expert, block_active, token_ids,
         row_wts):
    grid_spec = pltpu.PrefetchScalarGridSpec(
        num_scalar_prefetch=4,
        grid=(NB, NCHUNK),
        in_specs=[
            pl.BlockSpec((T, D), lambda m, n, be, ba, tid, wt: (0, 0)),
            pl.BlockSpec((1, BN, D), lambda m, n, be, ba, tid, wt: (be[m], n, 0)),
            pl.BlockSpec((1, BN, D), lambda m, n, be, ba, tid, wt: (be[m], n, 0)),
            pl.BlockSpec((1, D, BN), lambda m, n, be, ba, tid, wt: (be[m], 0, n)),
        ],
        out_specs=pl.BlockSpec((T, D), lambda m, n, be, ba, tid, wt: (0, 0)),
        scratch_shapes=[
            pltpu.VMEM((BM, D), jnp.float32),
            pltpu.VMEM((BM, D), jnp.float32),
        ],
    )
    return pl.pallas_call(
        _gmm_body,
        grid_spec=grid_spec,
        out_shape=jax.ShapeDtypeStruct((T, D), jnp.float32),
        compiler_params=pltpu.CompilerParams(
            dimension_semantics=("arbitrary", "arbitrary")),
    )(block_expert, block_active, token_ids, row_wts, xf, gate_w, up_w,
      down_w)


def kernel(x, router_w, gate_w, up_w, down_w):
    Bq, Tq, Dq = x.shape
    xf = x.reshape(Tq, Dq)

    ids, wts, aux = _router(xf, router_w)

    e_flat = ids.reshape(-1)
    onehot = jax.nn.one_hot(e_flat, E, dtype=jnp.int32)
    ranks_excl = jnp.cumsum(onehot, axis=0) - onehot
    rank = jnp.sum(ranks_excl * onehot, axis=1)
    counts = jnp.sum(onehot, axis=0)
    padded = ((counts + BM - 1) // BM) * BM
    ends = jnp.cumsum(padded)
    p_off = ends - padded
    pos = jnp.sum(onehot * p_off[None, :], axis=1) + rank
    tok = jnp.arange(K * T, dtype=jnp.int32) // K
    token_ids = jnp.zeros((NROWS,), jnp.int32).at[pos].set(tok)
    row_wts = jnp.zeros((NROWS,), jnp.float32).at[pos].set(wts.reshape(-1))
    total = ends[E - 1]
    starts = jnp.arange(NB, dtype=jnp.int32) * BM
    block_active = (starts < total).astype(jnp.int32)
    block_expert = jnp.minimum(
        jnp.sum((starts[:, None] >= ends[None, :]).astype(jnp.int32), axis=1),
        E - 1).astype(jnp.int32)

    out = _gmm(xf, gate_w, up_w, down_w, block_expert, block_active,
               token_ids, row_wts)

    return out.reshape(Bq, Tq, Dq), aux[0, 0]


# speedup vs baseline: 1.5530x; 1.1221x over previous
import jax
import jax.numpy as jnp
from jax.experimental import pallas as pl
from jax.experimental.pallas import tpu as pltpu

D = 2048
F = 8192
E = 8
K = 2
T = 2048

BM = 512
BN = 256
NCHUNK = F // BN
NB = (K * T) // BM + E
NROWS = NB * BM


def _router_body(x_ref, rw_ref, ids_ref, wts_ref, aux_ref):
    x = x_ref[...]
    rw = rw_ref[...]
    logits = jax.lax.dot_general(
        x, rw, (((1,), (1,)), ((), ())),
        preferred_element_type=jnp.float32)
    mx = jnp.max(logits, axis=1, keepdims=True)
    ex = jnp.exp(logits - mx)
    scores = ex / jnp.sum(ex, axis=1, keepdims=True)

    idx = jax.lax.broadcasted_iota(jnp.int32, scores.shape, 1)
    m1 = jnp.max(scores, axis=1, keepdims=True)
    i1 = jnp.min(jnp.where(scores == m1, idx, E), axis=1, keepdims=True)
    masked = jnp.where(idx == i1, -jnp.float32(1e30), scores)
    m2 = jnp.max(masked, axis=1, keepdims=True)
    i2 = jnp.min(jnp.where(masked == m2, idx, E), axis=1, keepdims=True)

    s = m1 + m2
    ids_ref[...] = jnp.concatenate([i1, i2], axis=1).astype(jnp.int32)
    wts_ref[...] = jnp.concatenate([m1 / s, m2 / s], axis=1)

    sel = (idx == i1) | (idx == i2)
    cnt = jnp.sum(sel.astype(jnp.float32), axis=0, keepdims=True)
    probs = jnp.sum(scores, axis=0, keepdims=True) / jnp.float32(T)
    aux_ref[...] = jnp.float32(E) * jnp.sum(
        probs * cnt / jnp.float32(T), axis=1, keepdims=True)


def _router(xf, router_w):
    return pl.pallas_call(
        _router_body,
        out_shape=[
            jax.ShapeDtypeStruct((T, K), jnp.int32),
            jax.ShapeDtypeStruct((T, K), jnp.float32),
            jax.ShapeDtypeStruct((1, 1), jnp.float32),
        ],
    )(xf, router_w)


def _gmm_body(be_ref, ba_ref, tid_ref, wt_ref, x_ref, g_ref, u_ref, d_ref,
              out_ref, xg_ref, acc_ref):
    m = pl.program_id(0)
    n = pl.program_id(1)
    active = ba_ref[m] > 0

    @pl.when((m == 0) & (n == 0))
    def _():
        out_ref[...] = jnp.zeros((T, D), jnp.float32)

    @pl.when(active & (n == 0))
    def _():
        def gather(i, c):
            t = tid_ref[m * BM + i]
            xg_ref[pl.ds(i, 1), :] = x_ref[pl.ds(t, 1), :]
            return c
        jax.lax.fori_loop(0, BM, gather, 0)

    @pl.when(active)
    def _():
        xg = xg_ref[...]
        g = jax.lax.dot_general(xg, g_ref[0], (((1,), (1,)), ((), ())),
                                preferred_element_type=jnp.float32)
        u = jax.lax.dot_general(xg, u_ref[0], (((1,), (1,)), ((), ())),
                                preferred_element_type=jnp.float32)
        h = (g * jax.lax.logistic(g)) * u
        y = jax.lax.dot_general(h, d_ref[0], (((1,), (1,)), ((), ())),
                                preferred_element_type=jnp.float32)

        @pl.when(n == 0)
        def _():
            acc_ref[...] = y

        @pl.when(n > 0)
        def _():
            acc_ref[...] = acc_ref[...] + y

    @pl.when(active & (n == NCHUNK - 1))
    def _():
        def scatter(i, c):
            t = tid_ref[m * BM + i]
            w = wt_ref[m * BM + i]
            out_ref[pl.ds(t, 1), :] = (out_ref[pl.ds(t, 1), :]
                                       + w * acc_ref[pl.ds(i, 1), :])
            return c
        jax.lax.fori_loop(0, BM, scatter, 0)


def _gmm(xf, gate_w, up_w, down_w, block_expert, block_active, token_ids,
         row_wts):
    grid_spec = pltpu.PrefetchScalarGridSpec(
        num_scalar_prefetch=4,
        grid=(NB, NCHUNK),
        in_specs=[
            pl.BlockSpec((T, D), lambda m, n, be, ba, tid, wt: (0, 0)),
            pl.BlockSpec((1, BN, D), lambda m, n, be, ba, tid, wt:
                         (be[m], jnp.where(ba[m] > 0, n, 0), 0)),
            pl.BlockSpec((1, BN, D), lambda m, n, be, ba, tid, wt:
                         (be[m], jnp.where(ba[m] > 0, n, 0), 0)),
            pl.BlockSpec((1, D, BN), lambda m, n, be, ba, tid, wt:
                         (be[m], 0, jnp.where(ba[m] > 0, n, 0))),
        ],
        out_specs=pl.BlockSpec((T, D), lambda m, n, be, ba, tid, wt: (0, 0)),
        scratch_shapes=[
            pltpu.VMEM((BM, D), jnp.float32),
            pltpu.VMEM((BM, D), jnp.float32),
        ],
    )
    return pl.pallas_call(
        _gmm_body,
        grid_spec=grid_spec,
        out_shape=jax.ShapeDtypeStruct((T, D), jnp.float32),
        compiler_params=pltpu.CompilerParams(
            dimension_semantics=("arbitrary", "arbitrary")),
    )(block_expert, block_active, token_ids, row_wts, xf, gate_w, up_w,
      down_w)


def kernel(x, router_w, gate_w, up_w, down_w):
    Bq, Tq, Dq = x.shape
    xf = x.reshape(Tq, Dq)

    ids, wts, aux = _router(xf, router_w)

    e_flat = ids.reshape(-1)
    onehot = jax.nn.one_hot(e_flat, E, dtype=jnp.int32)
    ranks_excl = jnp.cumsum(onehot, axis=0) - onehot
    rank = jnp.sum(ranks_excl * onehot, axis=1)
    counts = jnp.sum(onehot, axis=0)
    padded = ((counts + BM - 1) // BM) * BM
    ends = jnp.cumsum(padded)
    p_off = ends - padded
    pos = jnp.sum(onehot * p_off[None, :], axis=1) + rank
    tok = jnp.arange(K * T, dtype=jnp.int32) // K
    token_ids = jnp.zeros((NROWS,), jnp.int32).at[pos].set(tok)
    row_wts = jnp.zeros((NROWS,), jnp.float32).at[pos].set(wts.reshape(-1))
    total = ends[E - 1]
    starts = jnp.arange(NB, dtype=jnp.int32) * BM
    block_active = (starts < total).astype(jnp.int32)
    block_expert = jnp.minimum(
        jnp.sum((starts[:, None] >= ends[None, :]).astype(jnp.int32), axis=1),
        E - 1).astype(jnp.int32)

    out = _gmm(xf, gate_w, up_w, down_w, block_expert, block_active,
               token_ids, row_wts)

    return out.reshape(Bq, Tq, Dq), aux[0, 0]
